# baseline (speedup 1.0000x reference)
import functools

import jax
import jax.numpy as jnp
from jax import lax
from jax.experimental import pallas as pl
from jax.experimental.pallas import tpu as pltpu
from jax.experimental.pallas import tpu_sc as plsc

B = 16384
D = 128
NUM_CORES = 2
NUM_SUBCORES = 16
NW = NUM_CORES * NUM_SUBCORES
BPW = B // NW
NCHUNK = 4
CH = BPW // NCHUNK

_mesh = plsc.VectorSubcoreMesh(core_axis_name="c", subcore_axis_name="s")


@functools.partial(
    pl.kernel,
    mesh=_mesh,
    out_type=jax.ShapeDtypeStruct((B, D), jnp.float32),
    scratch_types=[
        pltpu.VMEM((BPW,), jnp.int32),
        pltpu.VMEM((NCHUNK, CH, D), jnp.float32),
        pltpu.SemaphoreType.DMA,
        pltpu.SemaphoreType.DMA,
    ],
)
def _gather_kernel(idx_hbm, table_hbm, out_hbm, idx_v, rows_v, gsem, psem):
    wid = lax.axis_index("s") * NUM_CORES + lax.axis_index("c")
    base = wid * BPW
    pltpu.sync_copy(idx_hbm.at[pl.ds(base, BPW)], idx_v)
    gathers = [
        pltpu.async_copy(
            table_hbm.at[idx_v.at[pl.ds(k * CH, CH)]], rows_v.at[k], gsem
        )
        for k in range(NCHUNK)
    ]
    puts = []
    for k in range(NCHUNK):
        gathers[k].wait()
        puts.append(
            pltpu.async_copy(
                rows_v.at[k], out_hbm.at[pl.ds(base + k * CH, CH)], psem
            )
        )
    for p in puts:
        p.wait()


def kernel(x, weight):
    return _gather_kernel(x.astype(jnp.int32), weight)


# speedup vs baseline: 1.0092x; 1.0092x over previous
import functools

import jax
import jax.numpy as jnp
from jax import lax
from jax.experimental import pallas as pl
from jax.experimental.pallas import tpu as pltpu
from jax.experimental.pallas import tpu_sc as plsc

B = 16384
D = 128
NUM_CORES = 2
NUM_SUBCORES = 16
NW = NUM_CORES * NUM_SUBCORES
BPW = B // NW
H = BPW // 2

_mesh = plsc.VectorSubcoreMesh(core_axis_name="c", subcore_axis_name="s")


@functools.partial(
    pl.kernel,
    mesh=_mesh,
    out_type=jax.ShapeDtypeStruct((B, D), jnp.float32),
    scratch_types=[
        pltpu.VMEM((BPW,), jnp.int32),
        pltpu.VMEM((2, H, D), jnp.float32),
        pltpu.SemaphoreType.DMA,
        pltpu.SemaphoreType.DMA,
    ],
)
def _gather_kernel(idx_hbm, table_hbm, out_hbm, idx_v, rows_v, gsem, psem):
    wid = lax.axis_index("s") * NUM_CORES + lax.axis_index("c")
    base = wid * BPW
    pltpu.sync_copy(idx_hbm.at[pl.ds(base, H)], idx_v.at[pl.ds(0, H)])
    g0 = pltpu.async_copy(table_hbm.at[idx_v.at[pl.ds(0, H)]], rows_v.at[0], gsem)
    pltpu.sync_copy(idx_hbm.at[pl.ds(base + H, H)], idx_v.at[pl.ds(H, H)])
    g1 = pltpu.async_copy(table_hbm.at[idx_v.at[pl.ds(H, H)]], rows_v.at[1], gsem)
    g0.wait()
    p0 = pltpu.async_copy(rows_v.at[0], out_hbm.at[pl.ds(base, H)], psem)
    g1.wait()
    p1 = pltpu.async_copy(rows_v.at[1], out_hbm.at[pl.ds(base + H, H)], psem)
    p0.wait()
    p1.wait()


def kernel(x, weight):
    return _gather_kernel(x.astype(jnp.int32), weight)
